# Initial kernel scaffold; baseline (speedup 1.0000x reference)
import jax
import jax.numpy as jnp
from jax.experimental import pallas as pl


def kernel(x, wt, fb):
    raise NotImplementedError("write your pallas kernel here")


# speedup vs baseline: 1.0508x; 1.0508x over previous
import jax
import jax.numpy as jnp
from jax.experimental import pallas as pl
from jax.experimental.pallas import tpu as pltpu

_BLOCK_B = 8192


def _fused_kernel(x_ref, wt_ref, fb_ref, o_ref):
    acc = jax.lax.dot_general(
        x_ref[...], wt_ref[...], (((1,), (0,)), ((), ())),
        preferred_element_type=jnp.float32)
    o_ref[...] = jnp.maximum(acc + fb_ref[...], 0.0).astype(o_ref.dtype)


def kernel(x, wt, fb):
    B, in_f = x.shape
    out_f = wt.shape[1]
    nb = pl.cdiv(B, _BLOCK_B)
    return pl.pallas_call(
        _fused_kernel,
        out_shape=jax.ShapeDtypeStruct((B, out_f), x.dtype),
        grid=(nb,),
        in_specs=[
            pl.BlockSpec((_BLOCK_B, in_f), lambda i: (i, 0)),
            pl.BlockSpec((in_f, out_f), lambda i: (0, 0)),
            pl.BlockSpec((1, out_f), lambda i: (0, 0)),
        ],
        out_specs=pl.BlockSpec((_BLOCK_B, out_f), lambda i: (i, 0)),
        compiler_params=pltpu.CompilerParams(
            dimension_semantics=("parallel",)),
    )(x, wt, fb)


# speedup vs baseline: 11.3912x; 10.8402x over previous
import jax
import jax.numpy as jnp
from jax.experimental import pallas as pl
from jax.experimental.pallas import tpu as pltpu

_BLOCK_N = 16384


def _t_kernel(x_ref, w_ref, fb_ref, o_ref):
    acc = jax.lax.dot_general(
        w_ref[...], x_ref[...], (((1,), (0,)), ((), ())),
        preferred_element_type=jnp.float32)
    o_ref[...] = jnp.maximum(acc + fb_ref[...], 0.0).astype(o_ref.dtype)


def kernel(x, wt, fb):
    B, in_f = x.shape
    out_f = wt.shape[1]
    xt = x.T
    w = wt.T
    fbt = fb.T
    nb = pl.cdiv(B, _BLOCK_N)
    yt = pl.pallas_call(
        _t_kernel,
        out_shape=jax.ShapeDtypeStruct((out_f, B), x.dtype),
        grid=(nb,),
        in_specs=[
            pl.BlockSpec((in_f, _BLOCK_N), lambda i: (0, i)),
            pl.BlockSpec((out_f, in_f), lambda i: (0, 0)),
            pl.BlockSpec((out_f, 1), lambda i: (0, 0)),
        ],
        out_specs=pl.BlockSpec((out_f, _BLOCK_N), lambda i: (0, i)),
        compiler_params=pltpu.CompilerParams(
            dimension_semantics=("parallel",)),
    )(xt, w, fbt)
    return yt.T
